# Initial kernel scaffold; baseline (speedup 1.0000x reference)
import jax
import jax.numpy as jnp
from jax.experimental import pallas as pl


def kernel(hms, pms_map, origin_shapes, pms_stats, u_base, shp_base, exp_base):
    raise NotImplementedError("write your pallas kernel here")


# speedup vs baseline: 1.6474x; 1.6474x over previous
import jax
import jax.numpy as jnp
import numpy as np
from jax.experimental import pallas as pl
from jax.experimental.pallas import tpu as pltpu

_B, _H, _W = 8, 160, 160
_N_S, _N_RT, _N_SHP, _N_EXP = 1, 11, 50, 29
_P = _N_S + _N_RT + _N_SHP + _N_EXP
_TOP_K = 1000
_NPAD = 1024
_N_OBJS = 200
_IOU = 0.5
_KPTS = 68
_RESIZE = np.array([160.0, 160.0], dtype=np.float32)


def _maxpool_kernel(x_ref, o_ref):
    xp = x_ref[0]
    c = xp[1:_H + 1, 1:_W + 1]
    m = c
    for dy in (0, 1, 2):
        for dx in (0, 1, 2):
            if dy == 1 and dx == 1:
                continue
            m = jnp.maximum(m, xp[dy:dy + _H, dx:dx + _W])
    o_ref[0] = jnp.where(m == c, c, 0.0)


def _decode_nms_kernel(scores_ref, aux_ref, params_ref, stats_ref, base_ref,
                       u_ref, bb_out, l0_out, l1_out, bb_buf, l0_buf, l1_buf):
    p = params_ref[0]
    mean = stats_ref[0:1, :]
    std = stats_ref[1:2, :]
    dp = p * std + mean
    se = dp[:, 12:91]
    v = jnp.dot(se, base_ref[...], preferred_element_type=jnp.float32)
    v = v + u_ref[...]
    vx = v[:, 0:68]
    vy = v[:, 68:136]
    vz = v[:, 136:204]
    s = dp[:, 0:1]
    l0 = s * (vx * dp[:, 1:2] + vy * dp[:, 2:3] + vz * dp[:, 3:4])
    l1 = s * (vx * dp[:, 5:6] + vy * dp[:, 6:7] + vz * dp[:, 7:8])
    a = aux_ref[0]
    L0 = l1 + a[:, 1:2]
    L1 = l0 + a[:, 2:3]
    l0_buf[...] = L0
    l1_buf[...] = L1
    tl0 = jnp.min(L0, axis=1, keepdims=True)
    tl1 = jnp.min(L1, axis=1, keepdims=True)
    br0 = jnp.max(L0, axis=1, keepdims=True)
    br1 = jnp.max(L1, axis=1, keepdims=True)
    area = (br0 - tl0) * (br1 - tl1)
    bb = jnp.concatenate(
        [tl0, tl1, br0, br1, a[:, 0:1], area, a[:, 3:4], jnp.zeros_like(tl0)],
        axis=1)
    bb_buf[...] = bb
    bbT = jnp.transpose(bb)
    y1v = bbT[0:1]
    x1v = bbT[1:2]
    y2v = bbT[2:3]
    x2v = bbT[3:4]
    areav = bbT[5:6]
    iota = jax.lax.broadcasted_iota(jnp.int32, (1, _NPAD), 1)
    work0 = scores_ref[0]

    def step(i, work):
        m = jnp.max(work)
        best = jnp.min(jnp.where(work == m, iota, _NPAD))
        row = bb_buf[pl.ds(best, 1), :]
        yy1 = jnp.maximum(y1v, row[0:1, 0:1])
        xx1 = jnp.maximum(x1v, row[0:1, 1:2])
        yy2 = jnp.minimum(y2v, row[0:1, 2:3])
        xx2 = jnp.minimum(x2v, row[0:1, 3:4])
        inter = jnp.maximum(yy2 - yy1, 0.0) * jnp.maximum(xx2 - xx1, 0.0)
        iou = inter / (row[0:1, 5:6] + areav - inter + 1e-8)
        sup = (iou > _IOU) | (iota == best)
        bb_out[0, pl.ds(i, 1), :] = row
        l0_out[0, pl.ds(i, 1), :] = l0_buf[pl.ds(best, 1), :]
        l1_out[0, pl.ds(i, 1), :] = l1_buf[pl.ds(best, 1), :]
        return jnp.where(sup, -jnp.inf, work)

    jax.lax.fori_loop(0, _N_OBJS, step, work0)


def kernel(hms, pms_map, origin_shapes, pms_stats, u_base, shp_base, exp_base):
    f32 = jnp.float32
    hms2 = hms[..., 0]
    hpad = jnp.pad(hms2, ((0, 0), (1, 1), (1, 1)),
                   constant_values=-jnp.inf)
    keep = pl.pallas_call(
        _maxpool_kernel,
        grid=(_B,),
        in_specs=[pl.BlockSpec((1, _H + 2, _W + 2), lambda b: (b, 0, 0))],
        out_specs=pl.BlockSpec((1, _H, _W), lambda b: (b, 0, 0)),
        out_shape=jax.ShapeDtypeStruct((_B, _H, _W), f32),
        compiler_params=pltpu.CompilerParams(
            dimension_semantics=("parallel",)),
    )(hpad)
    flat = keep.reshape(_B, _H * _W)
    topv, topi = jax.lax.top_k(flat, _TOP_K)
    rr = origin_shapes / jnp.asarray(_RESIZE)
    ys = (topi // _W).astype(f32)
    xs = (topi % _W).astype(f32)
    cy = ys * rr[:, 0:1]
    cx = xs * rr[:, 1:2]
    pms_flat = pms_map.reshape(_B, _H * _W, _P)
    params = jnp.take_along_axis(pms_flat, topi[..., None], axis=1)
    pad_n = _NPAD - _TOP_K
    params_pad = jnp.pad(params, ((0, 0), (0, pad_n), (0, 0)))
    scores_pad = jnp.pad(topv, ((0, 0), (0, pad_n)),
                         constant_values=-jnp.inf)
    zed = jnp.zeros((_B, _NPAD), f32)
    idxf = jnp.broadcast_to(jnp.arange(_NPAD, dtype=f32), (_B, _NPAD))
    aux = jnp.stack(
        [scores_pad,
         jnp.pad(cy, ((0, 0), (0, pad_n))),
         jnp.pad(cx, ((0, 0), (0, pad_n))),
         idxf, zed, zed, zed, zed], axis=-1)

    sb = shp_base.reshape(_KPTS, 3, _N_SHP)
    eb = exp_base.reshape(_KPTS, 3, _N_EXP)
    base_all = jnp.concatenate(
        [jnp.concatenate([sb[:, 0, :].T, sb[:, 1, :].T, sb[:, 2, :].T], axis=1),
         jnp.concatenate([eb[:, 0, :].T, eb[:, 1, :].T, eb[:, 2, :].T], axis=1)],
        axis=0)
    u3 = u_base[:, 0].reshape(_KPTS, 3)
    u_all = jnp.concatenate([u3[:, 0], u3[:, 1], u3[:, 2]]).reshape(1, 204)

    bb_sel, l0_sel, l1_sel = pl.pallas_call(
        _decode_nms_kernel,
        grid=(_B,),
        in_specs=[
            pl.BlockSpec((1, 1, _NPAD), lambda b: (b, 0, 0)),
            pl.BlockSpec((1, _NPAD, 8), lambda b: (b, 0, 0)),
            pl.BlockSpec((1, _NPAD, _P), lambda b: (b, 0, 0)),
            pl.BlockSpec((2, _P), lambda b: (0, 0)),
            pl.BlockSpec((79, 204), lambda b: (0, 0)),
            pl.BlockSpec((1, 204), lambda b: (0, 0)),
        ],
        out_specs=[
            pl.BlockSpec((1, _N_OBJS, 8), lambda b: (b, 0, 0)),
            pl.BlockSpec((1, _N_OBJS, _KPTS), lambda b: (b, 0, 0)),
            pl.BlockSpec((1, _N_OBJS, _KPTS), lambda b: (b, 0, 0)),
        ],
        out_shape=[
            jax.ShapeDtypeStruct((_B, _N_OBJS, 8), f32),
            jax.ShapeDtypeStruct((_B, _N_OBJS, _KPTS), f32),
            jax.ShapeDtypeStruct((_B, _N_OBJS, _KPTS), f32),
        ],
        scratch_shapes=[
            pltpu.VMEM((_NPAD, 8), f32),
            pltpu.VMEM((_NPAD, _KPTS), f32),
            pltpu.VMEM((_NPAD, _KPTS), f32),
        ],
        compiler_params=pltpu.CompilerParams(
            dimension_semantics=("arbitrary",)),
    )(scores_pad[:, None, :], aux, params_pad, pms_stats, base_all, u_all)

    out_bboxes = jnp.concatenate(
        [bb_sel[..., 0:5], jnp.zeros((_B, _N_OBJS, 1), f32)], axis=-1)
    out_lnmks = jnp.stack([l0_sel, l1_sel], axis=-1)

    sel = bb_sel[..., 6].astype(jnp.int32)
    psel = jnp.take_along_axis(params_pad, sel[..., None], axis=1)
    dsel = psel * pms_stats[1] + pms_stats[0]
    e0 = dsel[..., 1]
    e4 = dsel[..., 5]
    e8 = dsel[..., 9]
    e9 = dsel[..., 10]
    e10 = dsel[..., 11]
    sy = jnp.clip(-e8, -0.999, 0.999)
    yaw = jnp.arcsin(sy) * (180.0 / np.pi)
    cyw = jnp.cos(yaw)
    cyw = jnp.where(jnp.abs(cyw) < 1e-6, 1e-6, cyw)
    pitch = jnp.arctan2(e9 / cyw, e10 / cyw) * (180.0 / np.pi)
    roll = jnp.arctan2(e4 / cyw, e0 / cyw) * (180.0 / np.pi)
    pose = jnp.stack([pitch, yaw, roll], axis=-1)
    return out_bboxes, out_lnmks, pose


# speedup vs baseline: 4.6575x; 2.8271x over previous
import jax
import jax.numpy as jnp
import numpy as np
from jax.experimental import pallas as pl
from jax.experimental.pallas import tpu as pltpu

_B, _H, _W = 8, 160, 160
_N_S, _N_RT, _N_SHP, _N_EXP = 1, 11, 50, 29
_P = _N_S + _N_RT + _N_SHP + _N_EXP
_TOP_K = 1000
_NPAD = 1024
_N_OBJS = 200
_IOU = 0.5
_KPTS = 68
_RESIZE = np.array([160.0, 160.0], dtype=np.float32)


def _maxpool_kernel(x_ref, o_ref):
    xp = x_ref[0]
    c = xp[1:_H + 1, 1:_W + 1]
    m = c
    for dy in (0, 1, 2):
        for dx in (0, 1, 2):
            if dy == 1 and dx == 1:
                continue
            m = jnp.maximum(m, xp[dy:dy + _H, dx:dx + _W])
    o_ref[0] = jnp.where(m == c, c, 0.0)


def _decode_nms_kernel(scores_ref, aux_ref, params_ref, stats_ref, base_ref,
                       u_ref, bb_out, l0_out, l1_out, bb_buf, l0_buf, l1_buf):
    mean = stats_ref[0:1, :]
    std = stats_ref[1:2, :]
    rows = []
    for b in range(_B):
        p = params_ref[b]
        dp = p * std + mean
        se = dp[:, 12:91]
        v = jnp.dot(se, base_ref[...], preferred_element_type=jnp.float32)
        v = v + u_ref[...]
        vx = v[:, 0:68]
        vy = v[:, 68:136]
        vz = v[:, 136:204]
        s = dp[:, 0:1]
        l0 = s * (vx * dp[:, 1:2] + vy * dp[:, 2:3] + vz * dp[:, 3:4])
        l1 = s * (vx * dp[:, 5:6] + vy * dp[:, 6:7] + vz * dp[:, 7:8])
        a = aux_ref[b]
        L0 = l1 + a[:, 1:2]
        L1 = l0 + a[:, 2:3]
        l0_buf[b] = L0
        l1_buf[b] = L1
        tl0 = jnp.min(L0, axis=1, keepdims=True)
        tl1 = jnp.min(L1, axis=1, keepdims=True)
        br0 = jnp.max(L0, axis=1, keepdims=True)
        br1 = jnp.max(L1, axis=1, keepdims=True)
        area = (br0 - tl0) * (br1 - tl1)
        bb = jnp.concatenate(
            [tl0, tl1, br0, br1, a[:, 0:1], area, a[:, 3:4],
             jnp.zeros_like(tl0)], axis=1)
        bb_buf[b] = bb
        rows.append(jnp.transpose(bb))
    y1v = jnp.concatenate([t[0:1] for t in rows], axis=0)
    x1v = jnp.concatenate([t[1:2] for t in rows], axis=0)
    y2v = jnp.concatenate([t[2:3] for t in rows], axis=0)
    x2v = jnp.concatenate([t[3:4] for t in rows], axis=0)
    areav = jnp.concatenate([t[5:6] for t in rows], axis=0)
    iota = jax.lax.broadcasted_iota(jnp.int32, (_B, _NPAD), 1)
    work0 = scores_ref[...]

    def step(i, work):
        m = jnp.max(work, axis=1, keepdims=True)
        best = jnp.min(jnp.where(work == m, iota, _NPAD),
                       axis=1, keepdims=True)
        bm = (iota == best)
        bmf = bm.astype(jnp.float32)
        by1 = jnp.sum(y1v * bmf, axis=1, keepdims=True)
        bx1 = jnp.sum(x1v * bmf, axis=1, keepdims=True)
        by2 = jnp.sum(y2v * bmf, axis=1, keepdims=True)
        bx2 = jnp.sum(x2v * bmf, axis=1, keepdims=True)
        barea = jnp.sum(areav * bmf, axis=1, keepdims=True)
        yy1 = jnp.maximum(y1v, by1)
        xx1 = jnp.maximum(x1v, bx1)
        yy2 = jnp.minimum(y2v, by2)
        xx2 = jnp.minimum(x2v, bx2)
        inter = jnp.maximum(yy2 - yy1, 0.0) * jnp.maximum(xx2 - xx1, 0.0)
        iou = inter / (barea + areav - inter + 1e-8)
        sup = (iou > _IOU) | bm
        for b in range(_B):
            bsel = best[b, 0]
            bb_out[b, pl.ds(i, 1), :] = bb_buf[b, pl.ds(bsel, 1), :]
            l0_out[b, pl.ds(i, 1), :] = l0_buf[b, pl.ds(bsel, 1), :]
            l1_out[b, pl.ds(i, 1), :] = l1_buf[b, pl.ds(bsel, 1), :]
        return jnp.where(sup, -jnp.inf, work)

    jax.lax.fori_loop(0, _N_OBJS, step, work0)


def kernel(hms, pms_map, origin_shapes, pms_stats, u_base, shp_base, exp_base):
    f32 = jnp.float32
    hms2 = hms[..., 0]
    hpad = jnp.pad(hms2, ((0, 0), (1, 1), (1, 1)),
                   constant_values=-jnp.inf)
    keep = pl.pallas_call(
        _maxpool_kernel,
        grid=(_B,),
        in_specs=[pl.BlockSpec((1, _H + 2, _W + 2), lambda b: (b, 0, 0))],
        out_specs=pl.BlockSpec((1, _H, _W), lambda b: (b, 0, 0)),
        out_shape=jax.ShapeDtypeStruct((_B, _H, _W), f32),
        compiler_params=pltpu.CompilerParams(
            dimension_semantics=("parallel",)),
    )(hpad)
    flat = keep.reshape(_B, _H * _W)
    topv, topi = jax.lax.top_k(flat, _TOP_K)
    rr = origin_shapes / jnp.asarray(_RESIZE)
    ys = (topi // _W).astype(f32)
    xs = (topi % _W).astype(f32)
    cy = ys * rr[:, 0:1]
    cx = xs * rr[:, 1:2]
    pms_flat = pms_map.reshape(_B, _H * _W, _P)
    params = jnp.take_along_axis(pms_flat, topi[..., None], axis=1)
    pad_n = _NPAD - _TOP_K
    params_pad = jnp.pad(params, ((0, 0), (0, pad_n), (0, 0)))
    scores_pad = jnp.pad(topv, ((0, 0), (0, pad_n)),
                         constant_values=-jnp.inf)
    zed = jnp.zeros((_B, _NPAD), f32)
    idxf = jnp.broadcast_to(jnp.arange(_NPAD, dtype=f32), (_B, _NPAD))
    aux = jnp.stack(
        [scores_pad,
         jnp.pad(cy, ((0, 0), (0, pad_n))),
         jnp.pad(cx, ((0, 0), (0, pad_n))),
         idxf, zed, zed, zed, zed], axis=-1)

    sb = shp_base.reshape(_KPTS, 3, _N_SHP)
    eb = exp_base.reshape(_KPTS, 3, _N_EXP)
    base_all = jnp.concatenate(
        [jnp.concatenate([sb[:, 0, :].T, sb[:, 1, :].T, sb[:, 2, :].T], axis=1),
         jnp.concatenate([eb[:, 0, :].T, eb[:, 1, :].T, eb[:, 2, :].T], axis=1)],
        axis=0)
    u3 = u_base[:, 0].reshape(_KPTS, 3)
    u_all = jnp.concatenate([u3[:, 0], u3[:, 1], u3[:, 2]]).reshape(1, 204)

    bb_sel, l0_sel, l1_sel = pl.pallas_call(
        _decode_nms_kernel,
        out_shape=[
            jax.ShapeDtypeStruct((_B, _N_OBJS, 8), f32),
            jax.ShapeDtypeStruct((_B, _N_OBJS, _KPTS), f32),
            jax.ShapeDtypeStruct((_B, _N_OBJS, _KPTS), f32),
        ],
        scratch_shapes=[
            pltpu.VMEM((_B, _NPAD, 8), f32),
            pltpu.VMEM((_B, _NPAD, _KPTS), f32),
            pltpu.VMEM((_B, _NPAD, _KPTS), f32),
        ],
    )(scores_pad, aux, params_pad, pms_stats, base_all, u_all)

    out_bboxes = jnp.concatenate(
        [bb_sel[..., 0:5], jnp.zeros((_B, _N_OBJS, 1), f32)], axis=-1)
    out_lnmks = jnp.stack([l0_sel, l1_sel], axis=-1)

    sel = bb_sel[..., 6].astype(jnp.int32)
    psel = jnp.take_along_axis(params_pad, sel[..., None], axis=1)
    dsel = psel * pms_stats[1] + pms_stats[0]
    e0 = dsel[..., 1]
    e4 = dsel[..., 5]
    e8 = dsel[..., 9]
    e9 = dsel[..., 10]
    e10 = dsel[..., 11]
    sy = jnp.clip(-e8, -0.999, 0.999)
    yaw = jnp.arcsin(sy) * (180.0 / np.pi)
    cyw = jnp.cos(yaw)
    cyw = jnp.where(jnp.abs(cyw) < 1e-6, 1e-6, cyw)
    pitch = jnp.arctan2(e9 / cyw, e10 / cyw) * (180.0 / np.pi)
    roll = jnp.arctan2(e4 / cyw, e0 / cyw) * (180.0 / np.pi)
    pose = jnp.stack([pitch, yaw, roll], axis=-1)
    return out_bboxes, out_lnmks, pose
